# baseline (speedup 1.0000x reference)
import functools

import jax
import jax.numpy as jnp
from jax import lax
from jax.experimental import pallas as pl
from jax.experimental.pallas import tpu as pltpu
from jax.experimental.pallas import tpu_sc as plsc

N = 10000
E = 320000
D = 128

NC = 2
NS = 16
NW = NC * NS
K = 80
L = 16
NP = 10240
RPT = NP // NS
EPC = 4096
EP = EPC * K
CP1 = EPC // NW
K2 = 128
EPC2 = EP // K2
CP2 = EPC2 // NS

_mesh = plsc.VectorSubcoreMesh(
    core_axis_name="c", subcore_axis_name="s", num_cores=NC, num_subcores=NS)


def _pipeline(nchunks, base, eidx_hbm, idx4, table, rows0, rows1, acc,
              gsem0, gsem1, ssem0, ssem1, isem0, isem1, extra=None):
    rows = (rows0, rows1)
    gsem = (gsem0, gsem1)
    ssem = (ssem0, ssem1)
    isem = (isem0, isem1)

    def pf(k, slot):
        pltpu.async_copy(eidx_hbm.at[base + k], idx4.at[slot], isem[slot % 2])

    def pf_wait(k, slot):
        pltpu.make_async_copy(eidx_hbm.at[base + k], idx4.at[slot],
                              isem[slot % 2]).wait()

    def gth(slot, b):
        pltpu.async_copy(table.at[idx4.at[slot, 0]], rows[b], gsem[b])

    def proc(slot, b):
        if extra is not None:
            extra(slot)
        pltpu.make_async_copy(table.at[idx4.at[slot, 0]], rows[b],
                              gsem[b]).wait()
        pltpu.async_copy(rows[b], acc.at[idx4.at[slot, 1]], ssem[b], add=True)

    def sct_wait(slot, b):
        pltpu.make_async_copy(rows[b], acc.at[idx4.at[slot, 1]],
                              ssem[b]).wait()

    pltpu.sync_copy(eidx_hbm.at[base + 0], idx4.at[0])
    pltpu.sync_copy(eidx_hbm.at[base + 1], idx4.at[1])
    gth(0, 0)
    gth(1, 1)
    pf(2, 2)
    pf(3, 3)

    def body(q, carry):
        c = 4 * q
        proc(0, 0)
        proc(1, 1)
        sct_wait(0, 0)
        pf_wait(c + 2, 2)
        gth(2, 0)
        pf(c + 4, 0)
        sct_wait(1, 1)
        pf_wait(c + 3, 3)
        gth(3, 1)
        pf(c + 5, 1)
        proc(2, 0)
        proc(3, 1)
        sct_wait(2, 0)
        pf_wait(c + 4, 0)
        gth(0, 0)
        pf(c + 6, 2)
        sct_wait(3, 1)
        pf_wait(c + 5, 1)
        gth(1, 1)
        pf(c + 7, 3)
        return carry

    lax.fori_loop(0, nchunks // 4 - 1, body, 0)
    proc(0, 0)
    proc(1, 1)
    sct_wait(0, 0)
    pf_wait(nchunks - 2, 2)
    gth(2, 0)
    sct_wait(1, 1)
    pf_wait(nchunks - 1, 3)
    gth(3, 1)
    proc(2, 0)
    proc(3, 1)
    sct_wait(2, 0)
    sct_wait(3, 1)


@functools.partial(
    pl.kernel,
    out_type=(
        jax.ShapeDtypeStruct((NC, NP, D), jnp.float32),
        jax.ShapeDtypeStruct((NW, NP), jnp.float32),
        jax.ShapeDtypeStruct((NW, NP), jnp.float32),
    ),
    mesh=_mesh,
    compiler_params=pltpu.CompilerParams(needs_layout_passes=False),
    scratch_types=[
        pltpu.VMEM_SHARED((NP, D), jnp.float32),
        pltpu.VMEM((4, 2, K), jnp.int32),
        pltpu.VMEM((K, D), jnp.float32),
        pltpu.VMEM((K, D), jnp.float32),
        pltpu.VMEM((NP,), jnp.float32),
        pltpu.VMEM((NP,), jnp.float32),
        pltpu.SemaphoreType.DMA,
        pltpu.SemaphoreType.DMA,
        pltpu.SemaphoreType.DMA,
        pltpu.SemaphoreType.DMA,
        pltpu.SemaphoreType.DMA,
        pltpu.SemaphoreType.DMA,
    ],
)
def _sc_aggr(xp_hbm, ei_hbm, zrows_hbm, zflat_hbm,
             out, deg_out, cnt_out,
             acc, idx, rows0, rows1, deg_l, cnt_l,
             gsem0, gsem1, ssem0, ssem1, isem0, isem1):
    c = lax.axis_index("c")
    s = lax.axis_index("s")
    wid = c * NS + s
    pltpu.sync_copy(zrows_hbm, acc.at[pl.ds(s * RPT, RPT)])
    pltpu.sync_copy(zflat_hbm, deg_l)
    pltpu.sync_copy(zflat_hbm, cnt_l)
    plsc.subcore_barrier()

    ones = jnp.ones((L,), jnp.float32)

    def hist(b):
        for j in range(K // L):
            plsc.addupdate_scatter(cnt_l, [idx[b, 0, pl.ds(j * L, L)]], ones)
            plsc.addupdate_scatter(deg_l, [idx[b, 1, pl.ds(j * L, L)]], ones)

    _pipeline(CP1, wid * CP1, ei_hbm, idx, xp_hbm, rows0, rows1, acc,
              gsem0, gsem1, ssem0, ssem1, isem0, isem1, extra=hist)

    pltpu.sync_copy(deg_l, deg_out.at[wid])
    pltpu.sync_copy(cnt_l, cnt_out.at[wid])
    plsc.subcore_barrier()
    pltpu.sync_copy(acc.at[pl.ds(s * RPT, RPT)],
                    out.at[c, pl.ds(s * RPT, RPT)])


@functools.partial(
    pl.kernel,
    out_type=jax.ShapeDtypeStruct((NC, NP, D), jnp.float32),
    mesh=_mesh,
    compiler_params=pltpu.CompilerParams(needs_layout_passes=False),
    scratch_types=[
        pltpu.VMEM_SHARED((NP, D), jnp.float32),
        pltpu.VMEM((4, 2, K2), jnp.int32),
        pltpu.VMEM((K2, D), jnp.float32),
        pltpu.VMEM((K2, D), jnp.float32),
        pltpu.SemaphoreType.DMA,
        pltpu.SemaphoreType.DMA,
        pltpu.SemaphoreType.DMA,
        pltpu.SemaphoreType.DMA,
        pltpu.SemaphoreType.DMA,
        pltpu.SemaphoreType.DMA,
    ],
)
def _sc_gate(b_hbm, b2_hbm, ei_hbm, zrows_hbm, out,
             acc, idx, rows0, rows1, gsem0, gsem1, ssem0, ssem1,
             isem0, isem1):
    c = lax.axis_index("c")
    s = lax.axis_index("s")
    pltpu.sync_copy(zrows_hbm, acc.at[pl.ds(s * RPT, RPT)])
    plsc.subcore_barrier()

    @pl.when(c == 0)
    def _():
        _pipeline(CP2, s * CP2, ei_hbm, idx, b_hbm, rows0, rows1, acc,
                  gsem0, gsem1, ssem0, ssem1, isem0, isem1)

    @pl.when(c == 1)
    def _():
        _pipeline(CP2, s * CP2, ei_hbm, idx, b2_hbm, rows0, rows1, acc,
                  gsem0, gsem1, ssem0, ssem1, isem0, isem1)

    plsc.subcore_barrier()
    pltpu.sync_copy(acc.at[pl.ds(s * RPT, RPT)],
                    out.at[c, pl.ds(s * RPT, RPT)])


_BN = 10240


def _tc_mid_body(x_ref, ap_ref, dp_ref, wl_ref, bl_ref, wr_ref, qw_ref,
                 qb_ref, a_ref, b_ref, b2_ref):
    ap = ap_ref[...]
    agg = ap[0] + ap[1]
    deg = jnp.maximum(jnp.sum(dp_ref[...], axis=0), 1.0)[:, None]
    aggr = agg / deg
    x = x_ref[...]
    wl = wl_ref[...]
    wr = wr_ref[...]
    qw = qw_ref[...]
    dn = (((1,), (1,)), ((), ()))
    xc = lax.dot_general(aggr, wl, dn, preferred_element_type=jnp.float32)
    xc = xc + lax.dot_general(x, wr, dn, preferred_element_type=jnp.float32)
    xc = jnp.maximum(xc + bl_ref[...], 0.0)
    a = lax.dot_general(xc, qw[:, :D], dn, preferred_element_type=jnp.float32)
    b = lax.dot_general(xc, qw[:, D:], dn, preferred_element_type=jnp.float32)
    b = b + qb_ref[...]
    a_ref[...] = a
    b_ref[...] = b
    b2_ref[...] = b * b


def _tc_mid(x, aggr_part, deg_part, wl, bl, wr, qw, qb):
    grid = (NP // _BN,)
    return pl.pallas_call(
        _tc_mid_body,
        grid=grid,
        in_specs=[
            pl.BlockSpec((_BN, D), lambda i: (i, 0)),
            pl.BlockSpec((NC, _BN, D), lambda i: (0, i, 0)),
            pl.BlockSpec((NW, _BN), lambda i: (0, i)),
            pl.BlockSpec((D, D), lambda i: (0, 0)),
            pl.BlockSpec((1, D), lambda i: (0, 0)),
            pl.BlockSpec((D, D), lambda i: (0, 0)),
            pl.BlockSpec((D, 2 * D), lambda i: (0, 0)),
            pl.BlockSpec((1, D), lambda i: (0, 0)),
        ],
        out_specs=[
            pl.BlockSpec((_BN, D), lambda i: (i, 0)),
            pl.BlockSpec((_BN, D), lambda i: (i, 0)),
            pl.BlockSpec((_BN, D), lambda i: (i, 0)),
        ],
        out_shape=[
            jax.ShapeDtypeStruct((N, D), jnp.float32),
            jax.ShapeDtypeStruct((NP, D), jnp.float32),
            jax.ShapeDtypeStruct((NP, D), jnp.float32),
        ],
    )(x, aggr_part, deg_part, wl, bl, wr, qw, qb)


def _tc_out_body(a_ref, sp_ref, cp_ref, gg_ref):
    sp = sp_ref[...]
    sb = sp[0]
    sb2 = sp[1]
    cnt = jnp.sum(cp_ref[...], axis=0)[:, None]
    a = a_ref[...]
    mean_q = (cnt * a * a + 2.0 * a * sb + sb2) / jnp.maximum(cnt, 1.0)
    gg_ref[...] = jnp.tanh(mean_q)


def _tc_out(a, sb_part, cnt_part):
    grid = (NP // _BN,)
    return pl.pallas_call(
        _tc_out_body,
        grid=grid,
        in_specs=[
            pl.BlockSpec((_BN, D), lambda i: (i, 0)),
            pl.BlockSpec((NC, _BN, D), lambda i: (0, i, 0)),
            pl.BlockSpec((NW, _BN), lambda i: (0, i)),
        ],
        out_specs=pl.BlockSpec((_BN, D), lambda i: (i, 0)),
        out_shape=jax.ShapeDtypeStruct((N, D), jnp.float32),
    )(a, sb_part, cnt_part)


def kernel(X, edge_index, Wl, bl, Wr, Qw, Qb):
    src = edge_index[0]
    dst = edge_index[1]
    pad = N + (jnp.arange(EP - E, dtype=jnp.int32) % (NP - N))
    srcp = jnp.concatenate([src, pad])
    dstp = jnp.concatenate([dst, pad])
    ei1 = jnp.stack([srcp.reshape(EPC, K), dstp.reshape(EPC, K)], axis=1)
    ei2 = jnp.stack([dstp.reshape(EPC2, K2), srcp.reshape(EPC2, K2)], axis=1)
    xp = jnp.pad(X, ((0, NP - N), (0, 0)))
    zrows = jnp.zeros((RPT, D), jnp.float32)
    zflat = jnp.zeros((NP,), jnp.float32)

    aggr_part, deg_part, cnt_part = _sc_aggr(xp, ei1, zrows, zflat)
    a, b, b2 = _tc_mid(X, aggr_part, deg_part, Wl, bl.reshape(1, D), Wr, Qw,
                       Qb.reshape(1, D))
    sb_part = _sc_gate(b, b2, ei2, zrows)
    return _tc_out(a, sb_part, cnt_part)


# speedup vs baseline: 1.0447x; 1.0447x over previous
import functools

import jax
import jax.numpy as jnp
from jax import lax
from jax.experimental import pallas as pl
from jax.experimental.pallas import tpu as pltpu
from jax.experimental.pallas import tpu_sc as plsc

N = 10000
E = 320000
D = 128

NC = 2
NS = 16
NW = NC * NS
K = 80
L = 16
NP = 10240
RPT = NP // NS
EPC = 4096
EP = EPC * K
CP1 = EPC // NW
K2 = 128
EPC2 = EP // K2
CP2 = EPC2 // NS

_mesh = plsc.VectorSubcoreMesh(
    core_axis_name="c", subcore_axis_name="s", num_cores=NC, num_subcores=NS)


def _pipeline(nchunks, base, eidx_hbm, idx, table,
              rows0, rows1, acc, gsem0, gsem1, ssem0, ssem1, extra=None):
    rows = (rows0, rows1)
    gsem = (gsem0, gsem1)
    ssem = (ssem0, ssem1)

    def start(k, b, first):
        if not first:
            pltpu.make_async_copy(rows[b], acc.at[idx.at[b, 1]],
                                  ssem[b]).wait()
        pltpu.sync_copy(eidx_hbm.at[base + k], idx.at[b])
        pltpu.async_copy(table.at[idx.at[b, 0]], rows[b], gsem[b])

    def proc(k, b):
        if extra is not None:
            extra(b)
        pltpu.make_async_copy(table.at[idx.at[b, 0]], rows[b], gsem[b]).wait()
        pltpu.async_copy(rows[b], acc.at[idx.at[b, 1]], ssem[b], add=True)

    start(0, 0, True)
    start(1, 1, True)

    def body(p, carry):
        k = 2 * p
        proc(k, 0)
        proc(k + 1, 1)
        start(k + 2, 0, False)
        start(k + 3, 1, False)
        return carry

    lax.fori_loop(0, nchunks // 2 - 1, body, 0)
    proc(nchunks - 2, 0)
    proc(nchunks - 1, 1)
    pltpu.make_async_copy(rows0, acc.at[idx.at[0, 1]], ssem0).wait()
    pltpu.make_async_copy(rows1, acc.at[idx.at[1, 1]], ssem1).wait()


@functools.partial(
    pl.kernel,
    out_type=(
        jax.ShapeDtypeStruct((NC, NP, D), jnp.float32),
        jax.ShapeDtypeStruct((NW, NP), jnp.float32),
        jax.ShapeDtypeStruct((NW, NP), jnp.float32),
    ),
    mesh=_mesh,
    compiler_params=pltpu.CompilerParams(needs_layout_passes=False),
    scratch_types=[
        pltpu.VMEM_SHARED((NP, D), jnp.float32),
        pltpu.VMEM((2, 2, K), jnp.int32),
        pltpu.VMEM((K, D), jnp.float32),
        pltpu.VMEM((K, D), jnp.float32),
        pltpu.VMEM((NP,), jnp.float32),
        pltpu.VMEM((NP,), jnp.float32),
        pltpu.SemaphoreType.DMA,
        pltpu.SemaphoreType.DMA,
        pltpu.SemaphoreType.DMA,
        pltpu.SemaphoreType.DMA,
    ],
)
def _sc_aggr(xp_hbm, ei_hbm, zrows_hbm, zflat_hbm,
             out, deg_out, cnt_out,
             acc, idx, rows0, rows1, deg_l, cnt_l,
             gsem0, gsem1, ssem0, ssem1):
    c = lax.axis_index("c")
    s = lax.axis_index("s")
    wid = c * NS + s
    pltpu.sync_copy(zrows_hbm, acc.at[pl.ds(s * RPT, RPT)])
    pltpu.sync_copy(zflat_hbm, deg_l)
    pltpu.sync_copy(zflat_hbm, cnt_l)
    plsc.subcore_barrier()

    ones = jnp.ones((L,), jnp.float32)

    def hist(b):
        for j in range(K // L):
            plsc.addupdate_scatter(cnt_l, [idx[b, 0, pl.ds(j * L, L)]], ones)
            plsc.addupdate_scatter(deg_l, [idx[b, 1, pl.ds(j * L, L)]], ones)

    _pipeline(CP1, wid * CP1, ei_hbm, idx, xp_hbm,
              rows0, rows1, acc, gsem0, gsem1, ssem0, ssem1, extra=hist)

    pltpu.sync_copy(deg_l, deg_out.at[wid])
    pltpu.sync_copy(cnt_l, cnt_out.at[wid])
    plsc.subcore_barrier()
    pltpu.sync_copy(acc.at[pl.ds(s * RPT, RPT)],
                    out.at[c, pl.ds(s * RPT, RPT)])


@functools.partial(
    pl.kernel,
    out_type=jax.ShapeDtypeStruct((NC, NP, D), jnp.float32),
    mesh=_mesh,
    compiler_params=pltpu.CompilerParams(needs_layout_passes=False),
    scratch_types=[
        pltpu.VMEM_SHARED((NP, D), jnp.float32),
        pltpu.VMEM((2, 2, K2), jnp.int32),
        pltpu.VMEM((K2, D), jnp.float32),
        pltpu.VMEM((K2, D), jnp.float32),
        pltpu.SemaphoreType.DMA,
        pltpu.SemaphoreType.DMA,
        pltpu.SemaphoreType.DMA,
        pltpu.SemaphoreType.DMA,
    ],
)
def _sc_gate(b_hbm, b2_hbm, ei_hbm, zrows_hbm, out,
             acc, idx, rows0, rows1, gsem0, gsem1, ssem0, ssem1):
    c = lax.axis_index("c")
    s = lax.axis_index("s")
    pltpu.sync_copy(zrows_hbm, acc.at[pl.ds(s * RPT, RPT)])
    plsc.subcore_barrier()

    @pl.when(c == 0)
    def _():
        _pipeline(CP2, s * CP2, ei_hbm, idx, b_hbm,
                  rows0, rows1, acc, gsem0, gsem1, ssem0, ssem1)

    @pl.when(c == 1)
    def _():
        _pipeline(CP2, s * CP2, ei_hbm, idx, b2_hbm,
                  rows0, rows1, acc, gsem0, gsem1, ssem0, ssem1)

    plsc.subcore_barrier()
    pltpu.sync_copy(acc.at[pl.ds(s * RPT, RPT)],
                    out.at[c, pl.ds(s * RPT, RPT)])


_BN = 10240


def _tc_mid_body(x_ref, ap_ref, dp_ref, wl_ref, bl_ref, wr_ref, qw_ref,
                 qb_ref, a_ref, b_ref, b2_ref):
    ap = ap_ref[...]
    agg = ap[0] + ap[1]
    deg = jnp.maximum(jnp.sum(dp_ref[...], axis=0), 1.0)[:, None]
    aggr = agg / deg
    x = x_ref[...]
    wl = wl_ref[...]
    wr = wr_ref[...]
    qw = qw_ref[...]
    dn = (((1,), (1,)), ((), ()))
    xc = lax.dot_general(aggr, wl, dn, preferred_element_type=jnp.float32)
    xc = xc + lax.dot_general(x, wr, dn, preferred_element_type=jnp.float32)
    xc = jnp.maximum(xc + bl_ref[...], 0.0)
    a = lax.dot_general(xc, qw[:, :D], dn, preferred_element_type=jnp.float32)
    b = lax.dot_general(xc, qw[:, D:], dn, preferred_element_type=jnp.float32)
    b = b + qb_ref[...]
    a_ref[...] = a
    b_ref[...] = b
    b2_ref[...] = b * b


def _tc_mid(x, aggr_part, deg_part, wl, bl, wr, qw, qb):
    grid = (NP // _BN,)
    return pl.pallas_call(
        _tc_mid_body,
        grid=grid,
        in_specs=[
            pl.BlockSpec((_BN, D), lambda i: (i, 0)),
            pl.BlockSpec((NC, _BN, D), lambda i: (0, i, 0)),
            pl.BlockSpec((NW, _BN), lambda i: (0, i)),
            pl.BlockSpec((D, D), lambda i: (0, 0)),
            pl.BlockSpec((1, D), lambda i: (0, 0)),
            pl.BlockSpec((D, D), lambda i: (0, 0)),
            pl.BlockSpec((D, 2 * D), lambda i: (0, 0)),
            pl.BlockSpec((1, D), lambda i: (0, 0)),
        ],
        out_specs=[
            pl.BlockSpec((_BN, D), lambda i: (i, 0)),
            pl.BlockSpec((_BN, D), lambda i: (i, 0)),
            pl.BlockSpec((_BN, D), lambda i: (i, 0)),
        ],
        out_shape=[
            jax.ShapeDtypeStruct((N, D), jnp.float32),
            jax.ShapeDtypeStruct((NP, D), jnp.float32),
            jax.ShapeDtypeStruct((NP, D), jnp.float32),
        ],
    )(x, aggr_part, deg_part, wl, bl, wr, qw, qb)


def _tc_out_body(a_ref, sp_ref, cp_ref, gg_ref):
    sp = sp_ref[...]
    sb = sp[0]
    sb2 = sp[1]
    cnt = jnp.sum(cp_ref[...], axis=0)[:, None]
    a = a_ref[...]
    mean_q = (cnt * a * a + 2.0 * a * sb + sb2) / jnp.maximum(cnt, 1.0)
    gg_ref[...] = jnp.tanh(mean_q)


def _tc_out(a, sb_part, cnt_part):
    grid = (NP // _BN,)
    return pl.pallas_call(
        _tc_out_body,
        grid=grid,
        in_specs=[
            pl.BlockSpec((_BN, D), lambda i: (i, 0)),
            pl.BlockSpec((NC, _BN, D), lambda i: (0, i, 0)),
            pl.BlockSpec((NW, _BN), lambda i: (0, i)),
        ],
        out_specs=pl.BlockSpec((_BN, D), lambda i: (i, 0)),
        out_shape=jax.ShapeDtypeStruct((N, D), jnp.float32),
    )(a, sb_part, cnt_part)


def kernel(X, edge_index, Wl, bl, Wr, Qw, Qb):
    src = edge_index[0]
    dst = edge_index[1]
    pad = N + (jnp.arange(EP - E, dtype=jnp.int32) % (NP - N))
    srcp = jnp.concatenate([src, pad])
    dstp = jnp.concatenate([dst, pad])
    ei1 = jnp.stack([srcp.reshape(EPC, K), dstp.reshape(EPC, K)], axis=1)
    ei2 = jnp.stack([dstp.reshape(EPC2, K2), srcp.reshape(EPC2, K2)], axis=1)
    xp = jnp.pad(X, ((0, NP - N), (0, 0)))
    zrows = jnp.zeros((RPT, D), jnp.float32)
    zflat = jnp.zeros((NP,), jnp.float32)

    aggr_part, deg_part, cnt_part = _sc_aggr(xp, ei1, zrows, zflat)
    a, b, b2 = _tc_mid(X, aggr_part, deg_part, Wl, bl.reshape(1, D), Wr, Qw,
                       Qb.reshape(1, D))
    sb_part = _sc_gate(b, b2, ei2, zrows)
    return _tc_out(a, sb_part, cnt_part)
